# Initial kernel scaffold; baseline (speedup 1.0000x reference)
import jax
import jax.numpy as jnp
from jax.experimental import pallas as pl


def kernel(z, embedding):
    raise NotImplementedError("write your pallas kernel here")


# speedup vs baseline: 2.5659x; 2.5659x over previous
import jax
import jax.numpy as jnp
from jax.experimental import pallas as pl

_NUM_CODES = 1024


def _vq_argmin_kernel(z_ref, emb_ref, idx_ref):
    b = pl.program_id(0)
    zb = z_ref[0]
    e = emb_ref[...]
    en = jnp.sum(e * e, axis=1, keepdims=True)
    scores = jnp.dot(e, zb, preferred_element_type=jnp.float32)
    d = en - 2.0 * scores
    m = jnp.min(d, axis=0, keepdims=True)
    iota = jax.lax.broadcasted_iota(jnp.int32, d.shape, 0)
    idx = jnp.min(jnp.where(d == m, iota, _NUM_CODES), axis=0, keepdims=True)
    idx_ref[pl.ds(b, 1), :] = idx


def kernel(z, embedding):
    B, C, H, W = z.shape
    pix = H * W
    z3 = z.reshape(B, C, pix)
    idx = pl.pallas_call(
        _vq_argmin_kernel,
        grid=(B,),
        in_specs=[
            pl.BlockSpec((1, C, pix), lambda b: (b, 0, 0)),
            pl.BlockSpec((_NUM_CODES, C), lambda b: (0, 0)),
        ],
        out_specs=pl.BlockSpec((B, pix), lambda b: (0, 0)),
        out_shape=jax.ShapeDtypeStruct((B, pix), jnp.int32),
    )(z3, embedding)
    indices = idx.reshape(B, H, W)
    return (z, indices)


# speedup vs baseline: 2.6141x; 1.0188x over previous
import jax
import jax.numpy as jnp
from jax.experimental import pallas as pl
from jax.experimental.pallas import tpu as pltpu

_NUM_CODES = 1024
_CODE_DIM = 64


def _vq_kernel(z_ref, emb_ref, zout_ref, idx_ref, e2_ref, en_ref):
    b = pl.program_id(0)

    @pl.when(b == 0)
    def _init():
        e = emb_ref[...]
        e2_ref[...] = -2.0 * e
        en = jnp.sum(e * e, axis=1, keepdims=True)
        en_ref[...] = jnp.broadcast_to(en, en_ref.shape)

    zb = z_ref[0]
    scores = jnp.dot(e2_ref[...], zb, preferred_element_type=jnp.float32)
    d = scores + en_ref[:, 0:1]
    m = jnp.min(d, axis=0, keepdims=True)
    iota = jax.lax.broadcasted_iota(jnp.int32, d.shape, 0)
    idx = jnp.min(jnp.where(d == m, iota, _NUM_CODES), axis=0, keepdims=True)
    idx_ref[pl.ds(b, 1), :] = idx
    zout_ref[...] = z_ref[...]


def kernel(z, embedding):
    B, C, H, W = z.shape
    pix = H * W
    z3 = z.reshape(B, C, pix)
    zout, idx = pl.pallas_call(
        _vq_kernel,
        grid=(B,),
        in_specs=[
            pl.BlockSpec((1, C, pix), lambda b: (b, 0, 0)),
            pl.BlockSpec((_NUM_CODES, C), lambda b: (0, 0)),
        ],
        out_specs=[
            pl.BlockSpec((1, C, pix), lambda b: (b, 0, 0)),
            pl.BlockSpec((B, pix), lambda b: (0, 0)),
        ],
        out_shape=[
            jax.ShapeDtypeStruct((B, C, pix), jnp.float32),
            jax.ShapeDtypeStruct((B, pix), jnp.int32),
        ],
        scratch_shapes=[
            pltpu.VMEM((_NUM_CODES, _CODE_DIM), jnp.float32),
            pltpu.VMEM((_NUM_CODES, 128), jnp.float32),
        ],
    )(z3, embedding)
    return (zout.reshape(B, C, H, W), idx.reshape(B, H, W))


# speedup vs baseline: 3.3125x; 1.2672x over previous
import jax
import jax.numpy as jnp
from jax.experimental import pallas as pl
from jax.experimental.pallas import tpu as pltpu

_NUM_CODES = 1024
_CODE_DIM = 64


def _vq_kernel(z_ref, emb_ref, zout_ref, idx_ref, e2_ref, en_ref):
    b = pl.program_id(0)

    @pl.when(b == 0)
    def _init():
        e = emb_ref[...]
        e2_ref[...] = -2.0 * e
        en = jnp.sum(e * e, axis=1, keepdims=True)
        en_ref[...] = jnp.broadcast_to(en, en_ref.shape)

    zb = z_ref[0]
    pix = zb.shape[1]
    scores = jnp.dot(e2_ref[...], zb, preferred_element_type=jnp.float32)
    d = scores + en_ref[:, 0:1]
    mv = d[0:8, :]
    mi = jnp.zeros((8, pix), jnp.int32)
    for r in range(1, _NUM_CODES // 8):
        row = jax.lax.slice(d, (8 * r, 0), (8 * r + 8, pix))
        take = row < mv
        mv = jnp.where(take, row, mv)
        mi = jnp.where(take, r, mi)
    siota = jax.lax.broadcasted_iota(jnp.int32, (8, pix), 0)
    codes = mi * 8 + siota
    best = jnp.min(mv, axis=0, keepdims=True)
    idx = jnp.min(jnp.where(mv == best, codes, _NUM_CODES), axis=0,
                  keepdims=True)
    idx_ref[pl.ds(b, 1), :] = idx
    zout_ref[...] = z_ref[...]


def kernel(z, embedding):
    B, C, H, W = z.shape
    pix = H * W
    z3 = z.reshape(B, C, pix)
    zout, idx = pl.pallas_call(
        _vq_kernel,
        grid=(B,),
        in_specs=[
            pl.BlockSpec((1, C, pix), lambda b: (b, 0, 0)),
            pl.BlockSpec((_NUM_CODES, C), lambda b: (0, 0)),
        ],
        out_specs=[
            pl.BlockSpec((1, C, pix), lambda b: (b, 0, 0)),
            pl.BlockSpec((B, pix), lambda b: (0, 0)),
        ],
        out_shape=[
            jax.ShapeDtypeStruct((B, C, pix), jnp.float32),
            jax.ShapeDtypeStruct((B, pix), jnp.int32),
        ],
        scratch_shapes=[
            pltpu.VMEM((_NUM_CODES, _CODE_DIM), jnp.float32),
            pltpu.VMEM((_NUM_CODES, 128), jnp.float32),
        ],
    )(z3, embedding)
    return (zout.reshape(B, C, H, W), idx.reshape(B, H, W))


# speedup vs baseline: 3.7166x; 1.1220x over previous
import jax
import jax.numpy as jnp
from jax.experimental import pallas as pl
from jax.experimental.pallas import tpu as pltpu

_NUM_CODES = 1024
_CODE_DIM = 64
_BPS = 2


def _vq_kernel(z_ref, emb_ref, zout_ref, idx_ref, e2_ref, en_ref):
    g = pl.program_id(0)

    @pl.when(g == 0)
    def _init():
        e = emb_ref[...]
        e2_ref[...] = -2.0 * e
        en = jnp.sum(e * e, axis=1, keepdims=True)
        en_ref[...] = jnp.broadcast_to(en, en_ref.shape)

    for j in range(_BPS):
        zb = z_ref[j]
        pix = zb.shape[1]
        scores = jnp.dot(e2_ref[...], zb, preferred_element_type=jnp.float32)
        d = scores + en_ref[:, 0:1]
        mv = d[0:8, :]
        mi = jnp.zeros((8, pix), jnp.int32)
        for r in range(1, _NUM_CODES // 8):
            row = jax.lax.slice(d, (8 * r, 0), (8 * r + 8, pix))
            take = row < mv
            mv = jnp.where(take, row, mv)
            mi = jnp.where(take, r, mi)
        siota = jax.lax.broadcasted_iota(jnp.int32, (8, pix), 0)
        codes = mi * 8 + siota
        best = jnp.min(mv, axis=0, keepdims=True)
        idx = jnp.min(jnp.where(mv == best, codes, _NUM_CODES), axis=0,
                      keepdims=True)
        idx_ref[j, 0:1, :] = idx
    zout_ref[...] = z_ref[...]


def kernel(z, embedding):
    B, C, H, W = z.shape
    pix = H * W
    z3 = z.reshape(B, C, pix)
    zout, idx = pl.pallas_call(
        _vq_kernel,
        grid=(B // _BPS,),
        in_specs=[
            pl.BlockSpec((_BPS, C, pix), lambda g: (g, 0, 0)),
            pl.BlockSpec((_NUM_CODES, C), lambda g: (0, 0)),
        ],
        out_specs=[
            pl.BlockSpec((_BPS, C, pix), lambda g: (g, 0, 0)),
            pl.BlockSpec((_BPS, 1, pix), lambda g: (g, 0, 0)),
        ],
        out_shape=[
            jax.ShapeDtypeStruct((B, C, pix), jnp.float32),
            jax.ShapeDtypeStruct((B, 1, pix), jnp.int32),
        ],
        scratch_shapes=[
            pltpu.VMEM((_NUM_CODES, _CODE_DIM), jnp.float32),
            pltpu.VMEM((_NUM_CODES, 128), jnp.float32),
        ],
    )(z3, embedding)
    return (zout.reshape(B, C, H, W), idx.reshape(B, H, W))


# speedup vs baseline: 3.8168x; 1.0269x over previous
import jax
import jax.numpy as jnp
from jax.experimental import pallas as pl
from jax.experimental.pallas import tpu as pltpu

_NUM_CODES = 1024
_CODE_DIM = 64
_BPS = 4


def _vq_kernel(z_ref, emb_ref, zout_ref, idx_ref, e2_ref, en_ref):
    g = pl.program_id(0)

    @pl.when(g == 0)
    def _init():
        e = emb_ref[...]
        e2_ref[...] = -2.0 * e
        en = jnp.sum(e * e, axis=1, keepdims=True)
        en_ref[...] = jnp.broadcast_to(en, en_ref.shape)

    for j in range(_BPS):
        zb = z_ref[j]
        pix = zb.shape[1]
        scores = jnp.dot(e2_ref[...], zb, preferred_element_type=jnp.float32)
        d = scores + en_ref[:, 0:1]
        mv = d[0:8, :]
        mi = jnp.zeros((8, pix), jnp.int32)
        for r in range(1, _NUM_CODES // 8):
            row = jax.lax.slice(d, (8 * r, 0), (8 * r + 8, pix))
            take = row < mv
            mv = jnp.where(take, row, mv)
            mi = jnp.where(take, r, mi)
        siota = jax.lax.broadcasted_iota(jnp.int32, (8, pix), 0)
        codes = mi * 8 + siota
        best = jnp.min(mv, axis=0, keepdims=True)
        idx = jnp.min(jnp.where(mv == best, codes, _NUM_CODES), axis=0,
                      keepdims=True)
        idx_ref[j, 0:1, :] = idx
    zout_ref[...] = z_ref[...]


def kernel(z, embedding):
    B, C, H, W = z.shape
    pix = H * W
    z3 = z.reshape(B, C, pix)
    zout, idx = pl.pallas_call(
        _vq_kernel,
        grid=(B // _BPS,),
        in_specs=[
            pl.BlockSpec((_BPS, C, pix), lambda g: (g, 0, 0)),
            pl.BlockSpec((_NUM_CODES, C), lambda g: (0, 0)),
        ],
        out_specs=[
            pl.BlockSpec((_BPS, C, pix), lambda g: (g, 0, 0)),
            pl.BlockSpec((_BPS, 1, pix), lambda g: (g, 0, 0)),
        ],
        out_shape=[
            jax.ShapeDtypeStruct((B, C, pix), jnp.float32),
            jax.ShapeDtypeStruct((B, 1, pix), jnp.int32),
        ],
        scratch_shapes=[
            pltpu.VMEM((_NUM_CODES, _CODE_DIM), jnp.float32),
            pltpu.VMEM((_NUM_CODES, 128), jnp.float32),
        ],
    )(z3, embedding)
    return (zout.reshape(B, C, H, W), idx.reshape(B, H, W))
